# baseline (speedup 1.0000x reference)
import functools

import jax
import jax.numpy as jnp
from jax import lax
from jax.experimental import pallas as pl
from jax.experimental.pallas import tpu as pltpu
from jax.experimental.pallas import tpu_sc as plsc

N = 320000
D = 128


TC_BLK = 10000


def _tc_body(x_ref, w_ref, b_ref, s_ref, a_ref, bb_ref, c_ref):
    x = x_ref[...]
    s_ref[...] = jnp.dot(x, w_ref[0], preferred_element_type=jnp.float32) + b_ref[...]
    a_ref[...] = jnp.dot(x, w_ref[1], preferred_element_type=jnp.float32)
    bb_ref[...] = jnp.dot(x, w_ref[2], preferred_element_type=jnp.float32)
    c_ref[...] = jnp.dot(x, w_ref[3], preferred_element_type=jnp.float32)


def _tc_transform(x, w_stacked, b_total):
    row_spec = pl.BlockSpec((TC_BLK, D), lambda i: (i, 0))
    return pl.pallas_call(
        _tc_body,
        grid=(N // TC_BLK,),
        in_specs=[
            row_spec,
            pl.BlockSpec((4, D, D), lambda i: (0, 0, 0)),
            pl.BlockSpec((1, D), lambda i: (0, 0)),
        ],
        out_specs=[row_spec, row_spec, row_spec, row_spec],
        out_shape=[
            jax.ShapeDtypeStruct((N, D), jnp.float32),
            jax.ShapeDtypeStruct((N, D), jnp.float32),
            jax.ShapeDtypeStruct((N, D), jnp.float32),
            jax.ShapeDtypeStruct((N, D), jnp.float32),
        ],
    )(x, w_stacked, b_total)


NC = 2
NS = 16
NW = NC * NS
PW = N // NW
R = 40
NCHUNK = PW // R
NPAIR = NCHUNK // 2


def _sc_body(s_hbm, a_hbm, b_hbm, c_hbm, ix_hbm, out_hbm,
             s0, a0, b0, c0, o0, s1, a1, b1, c1, o1,
             ix0, ix1,
             semi0, semo0, semx0, semi1, semo1, semx1):
    wid = lax.axis_index("s") * NC + lax.axis_index("c")
    base0 = wid * PW
    bufs = ((s0, a0, b0, c0, o0, ix0, semi0, semo0, semx0),
            (s1, a1, b1, c1, o1, ix1, semi1, semo1, semx1))

    def idx_copies(ci, buf):
        ix_v, semx = buf[5], buf[8]
        return (pltpu.make_async_copy(ix_hbm.at[wid, ci], ix_v, semx),)

    def in_copies(ci, buf):
        s_v, a_v, b_v, c_v = buf[:4]
        ix_v, semi = buf[5], buf[6]
        base = base0 + ci * R
        return (
            pltpu.make_async_copy(a_hbm.at[ix_v.at[0]], a_v, semi),
            pltpu.make_async_copy(b_hbm.at[ix_v.at[1]], b_v, semi),
            pltpu.make_async_copy(c_hbm.at[ix_v.at[2]], c_v, semi),
            pltpu.make_async_copy(s_hbm.at[pl.ds(base, R)], s_v, semi),
        )

    def out_copy(ci, buf):
        o_v, semo = buf[4], buf[7]
        return pltpu.make_async_copy(o_v, out_hbm.at[pl.ds(base0 + ci * R, R)], semo)

    def compute(buf):
        s_v, a_v, b_v, c_v, o_v = buf[:5]

        def pair_rows(r2, c2):
            r = 2 * r2
            for k in (0, 1):
                for j in range(D // 16):
                    sl = pl.ds(j * 16, 16)
                    v = (s_v[r + k, sl] + a_v[r + k, sl]
                         + b_v[r + k, sl] + c_v[r + k, sl])
                    o_v[r + k, sl] = jnp.maximum(v, 0.0)
            return c2

        lax.fori_loop(0, R // 2, pair_rows, 0, unroll=False)

    for sub in (0, 1):
        for d in idx_copies(sub, bufs[sub]):
            d.start()
    for sub in (0, 1):
        for d in idx_copies(sub, bufs[sub]):
            d.wait()
        for d in in_copies(sub, bufs[sub]):
            d.start()

    def pair_body(t, carry):
        for sub in (0, 1):
            buf = bufs[sub]
            ci = 2 * t + sub
            for d in in_copies(ci, buf):
                d.wait()

            @pl.when(ci + 2 < NCHUNK)
            def _():
                for d in idx_copies(ci + 2, buf):
                    d.start()

            @pl.when(t > 0)
            def _():
                out_copy(ci - 2, buf).wait()

            compute(buf)
            out_copy(ci, buf).start()

            @pl.when(ci + 2 < NCHUNK)
            def _():
                for d in idx_copies(ci + 2, buf):
                    d.wait()
                for d in in_copies(ci + 2, buf):
                    d.start()
        return carry

    lax.fori_loop(0, NPAIR, pair_body, 0, unroll=False)
    out_copy(NCHUNK - 2, bufs[0]).wait()
    out_copy(NCHUNK - 1, bufs[1]).wait()


def _sc_combine(s, a, b, c, idx_n, idx_p, idx_m):
    mesh = plsc.VectorSubcoreMesh(core_axis_name="c", subcore_axis_name="s")
    rows_f32 = pltpu.VMEM((R, D), jnp.float32)
    idx_t = pltpu.VMEM((3, R), jnp.int32)
    fn = pl.kernel(
        _sc_body,
        out_type=jax.ShapeDtypeStruct((N, D), jnp.float32),
        mesh=mesh,
        scratch_types=(
            [rows_f32] * 10
            + [idx_t] * 2
            + [pltpu.SemaphoreType.DMA] * 6
        ),
    )
    idx_all = jnp.stack(
        [idx_n.reshape(NW, NCHUNK, R),
         idx_p.reshape(NW, NCHUNK, R),
         idx_m.reshape(NW, NCHUNK, R)], axis=2)
    return fn(s, a, b, c, idx_all)


def kernel(features, next_indices, prev_indices, mate_indices, face_indices,
           W_self, b_self, W_next, b_next, W_prev, b_prev, W_mate, b_mate):
    del face_indices

    w_stacked = jnp.stack([W_self.T, W_next.T, W_prev.T, W_mate.T])
    b_total = (b_self + b_next + b_prev + b_mate).reshape(1, D)
    s, a, b, c = _tc_transform(features, w_stacked, b_total)
    return _sc_combine(
        s, a, b, c,
        next_indices.astype(jnp.int32),
        prev_indices.astype(jnp.int32),
        mate_indices.astype(jnp.int32),
    )


# speedup vs baseline: 1.0608x; 1.0608x over previous
import functools

import jax
import jax.numpy as jnp
from jax import lax
from jax.experimental import pallas as pl
from jax.experimental.pallas import tpu as pltpu
from jax.experimental.pallas import tpu_sc as plsc

N = 320000
D = 128


TC_BLK = 10000


def _tc_body(x_ref, w_ref, b_ref, s_ref, a_ref, bb_ref, c_ref):
    x = x_ref[...]
    s_ref[...] = jnp.dot(x, w_ref[0], preferred_element_type=jnp.float32) + b_ref[...]
    a_ref[...] = jnp.dot(x, w_ref[1], preferred_element_type=jnp.float32)
    bb_ref[...] = jnp.dot(x, w_ref[2], preferred_element_type=jnp.float32)
    c_ref[...] = jnp.dot(x, w_ref[3], preferred_element_type=jnp.float32)


def _tc_transform(x, w_stacked, b_total):
    row_spec = pl.BlockSpec((TC_BLK, D), lambda i: (i, 0))
    return pl.pallas_call(
        _tc_body,
        grid=(N // TC_BLK,),
        in_specs=[
            row_spec,
            pl.BlockSpec((4, D, D), lambda i: (0, 0, 0)),
            pl.BlockSpec((1, D), lambda i: (0, 0)),
        ],
        out_specs=[row_spec, row_spec, row_spec, row_spec],
        out_shape=[
            jax.ShapeDtypeStruct((N, D), jnp.float32),
            jax.ShapeDtypeStruct((N, D), jnp.float32),
            jax.ShapeDtypeStruct((N, D), jnp.float32),
            jax.ShapeDtypeStruct((N, D), jnp.float32),
        ],
    )(x, w_stacked, b_total)


NC = 2
NS = 16
NW = NC * NS
PW = N // NW
R = 40
NCHUNK = PW // R
NPAIR = NCHUNK // 2


def _sc_body(s_hbm, a_hbm, b_hbm, c_hbm, in_hbm, ip_hbm, im_hbm, out_hbm,
             s0, a0, b0, c0, o0, s1, a1, b1, c1, o1,
             in0, ip0, im0, in1, ip1, im1,
             semi0, semo0, semx0, semi1, semo1, semx1):
    wid = lax.axis_index("s") * NC + lax.axis_index("c")
    base0 = wid * PW
    bufs = ((s0, a0, b0, c0, o0, in0, ip0, im0, semi0, semo0, semx0),
            (s1, a1, b1, c1, o1, in1, ip1, im1, semi1, semo1, semx1))

    def idx_copies(ci, buf):
        in_v, ip_v, im_v, semx = buf[5], buf[6], buf[7], buf[10]
        return (
            pltpu.make_async_copy(in_hbm.at[wid, ci], in_v, semx),
            pltpu.make_async_copy(ip_hbm.at[wid, ci], ip_v, semx),
            pltpu.make_async_copy(im_hbm.at[wid, ci], im_v, semx),
        )

    def in_copies(ci, buf):
        s_v, a_v, b_v, c_v = buf[:4]
        in_v, ip_v, im_v, semi = buf[5], buf[6], buf[7], buf[8]
        base = base0 + ci * R
        return (
            pltpu.make_async_copy(a_hbm.at[in_v], a_v, semi),
            pltpu.make_async_copy(b_hbm.at[ip_v], b_v, semi),
            pltpu.make_async_copy(c_hbm.at[im_v], c_v, semi),
            pltpu.make_async_copy(s_hbm.at[pl.ds(base, R)], s_v, semi),
        )

    def out_copy(ci, buf):
        o_v, semo = buf[4], buf[9]
        return pltpu.make_async_copy(o_v, out_hbm.at[pl.ds(base0 + ci * R, R)], semo)

    def compute(buf):
        s_v, a_v, b_v, c_v, o_v = buf[:5]

        def pair_rows(r2, c2):
            r = 2 * r2
            for k in (0, 1):
                for j in range(D // 16):
                    sl = pl.ds(j * 16, 16)
                    v = (s_v[r + k, sl] + a_v[r + k, sl]
                         + b_v[r + k, sl] + c_v[r + k, sl])
                    o_v[r + k, sl] = jnp.maximum(v, 0.0)
            return c2

        lax.fori_loop(0, R // 2, pair_rows, 0, unroll=False)

    for sub in (0, 1):
        for d in idx_copies(sub, bufs[sub]):
            d.start()
    for sub in (0, 1):
        for d in idx_copies(sub, bufs[sub]):
            d.wait()
        for d in in_copies(sub, bufs[sub]):
            d.start()

    def pair_body(t, carry):
        for sub in (0, 1):
            buf = bufs[sub]
            ci = 2 * t + sub
            for d in in_copies(ci, buf):
                d.wait()

            @pl.when(ci + 2 < NCHUNK)
            def _():
                for d in idx_copies(ci + 2, buf):
                    d.start()

            @pl.when(t > 0)
            def _():
                out_copy(ci - 2, buf).wait()

            compute(buf)
            out_copy(ci, buf).start()

            @pl.when(ci + 2 < NCHUNK)
            def _():
                for d in idx_copies(ci + 2, buf):
                    d.wait()
                for d in in_copies(ci + 2, buf):
                    d.start()
        return carry

    lax.fori_loop(0, NPAIR, pair_body, 0, unroll=False)
    out_copy(NCHUNK - 2, bufs[0]).wait()
    out_copy(NCHUNK - 1, bufs[1]).wait()


def _sc_combine(s, a, b, c, idx_n, idx_p, idx_m):
    mesh = plsc.VectorSubcoreMesh(core_axis_name="c", subcore_axis_name="s")
    rows_f32 = pltpu.VMEM((R, D), jnp.float32)
    idx_t = pltpu.VMEM((R,), jnp.int32)
    fn = pl.kernel(
        _sc_body,
        out_type=jax.ShapeDtypeStruct((N, D), jnp.float32),
        mesh=mesh,
        scratch_types=(
            [rows_f32] * 10
            + [idx_t] * 6
            + [pltpu.SemaphoreType.DMA] * 6
        ),
    )
    return fn(
        s, a, b, c,
        idx_n.reshape(NW, NCHUNK, R),
        idx_p.reshape(NW, NCHUNK, R),
        idx_m.reshape(NW, NCHUNK, R),
    )


def kernel(features, next_indices, prev_indices, mate_indices, face_indices,
           W_self, b_self, W_next, b_next, W_prev, b_prev, W_mate, b_mate):
    del face_indices

    w_stacked = jnp.stack([W_self.T, W_next.T, W_prev.T, W_mate.T])
    b_total = (b_self + b_next + b_prev + b_mate).reshape(1, D)
    s, a, b, c = _tc_transform(features, w_stacked, b_total)
    return _sc_combine(
        s, a, b, c,
        next_indices.astype(jnp.int32),
        prev_indices.astype(jnp.int32),
        mate_indices.astype(jnp.int32),
    )


# speedup vs baseline: 1.0992x; 1.0362x over previous
import functools

import jax
import jax.numpy as jnp
from jax import lax
from jax.experimental import pallas as pl
from jax.experimental.pallas import tpu as pltpu
from jax.experimental.pallas import tpu_sc as plsc

N = 320000
D = 128


TC_BLK = 10000


def _tc_body(x_ref, w_ref, b_ref, s_ref, a_ref, bb_ref, c_ref):
    x = x_ref[...]
    s_ref[...] = jnp.dot(x, w_ref[0], preferred_element_type=jnp.float32) + b_ref[...]
    a_ref[...] = jnp.dot(x, w_ref[1], preferred_element_type=jnp.float32)
    bb_ref[...] = jnp.dot(x, w_ref[2], preferred_element_type=jnp.float32)
    c_ref[...] = jnp.dot(x, w_ref[3], preferred_element_type=jnp.float32)


def _tc_transform(x, w_stacked, b_total):
    row_spec = pl.BlockSpec((TC_BLK, D), lambda i: (i, 0))
    return pl.pallas_call(
        _tc_body,
        grid=(N // TC_BLK,),
        in_specs=[
            row_spec,
            pl.BlockSpec((4, D, D), lambda i: (0, 0, 0)),
            pl.BlockSpec((1, D), lambda i: (0, 0)),
        ],
        out_specs=[row_spec, row_spec, row_spec, row_spec],
        out_shape=[
            jax.ShapeDtypeStruct((N, D), jnp.float32),
            jax.ShapeDtypeStruct((N, D), jnp.float32),
            jax.ShapeDtypeStruct((N, D), jnp.float32),
            jax.ShapeDtypeStruct((N, D), jnp.float32),
        ],
    )(x, w_stacked, b_total)


NC = 2
NS = 16
NW = NC * NS
PW = N // NW
R = 40
NCHUNK = PW // R
NPAIR = NCHUNK // 2


def _sc_body(s_hbm, a_hbm, b_hbm, c_hbm, in_hbm, ip_hbm, im_hbm, out_hbm,
             s0, a0, b0, c0, o0, s1, a1, b1, c1, o1,
             in0, ip0, im0, in1, ip1, im1,
             semi0, semo0, semx0, semi1, semo1, semx1):
    wid = lax.axis_index("s") * NC + lax.axis_index("c")
    base0 = wid * PW
    bufs = ((s0, a0, b0, c0, o0, in0, ip0, im0, semi0, semo0, semx0),
            (s1, a1, b1, c1, o1, in1, ip1, im1, semi1, semo1, semx1))

    def idx_copies(ci, buf):
        in_v, ip_v, im_v, semx = buf[5], buf[6], buf[7], buf[10]
        base = base0 + ci * R
        return (
            pltpu.make_async_copy(in_hbm.at[pl.ds(base, R)], in_v, semx),
            pltpu.make_async_copy(ip_hbm.at[pl.ds(base, R)], ip_v, semx),
            pltpu.make_async_copy(im_hbm.at[pl.ds(base, R)], im_v, semx),
        )

    def in_copies(ci, buf):
        s_v, a_v, b_v, c_v = buf[:4]
        in_v, ip_v, im_v, semi = buf[5], buf[6], buf[7], buf[8]
        base = base0 + ci * R
        return (
            pltpu.make_async_copy(a_hbm.at[in_v], a_v, semi),
            pltpu.make_async_copy(b_hbm.at[ip_v], b_v, semi),
            pltpu.make_async_copy(c_hbm.at[im_v], c_v, semi),
            pltpu.make_async_copy(s_hbm.at[pl.ds(base, R)], s_v, semi),
        )

    def out_copy(ci, buf):
        o_v, semo = buf[4], buf[9]
        return pltpu.make_async_copy(o_v, out_hbm.at[pl.ds(base0 + ci * R, R)], semo)

    def compute(buf):
        s_v, a_v, b_v, c_v, o_v = buf[:5]

        def pair_rows(r2, c2):
            r = 2 * r2
            for k in (0, 1):
                for j in range(D // 16):
                    sl = pl.ds(j * 16, 16)
                    v = (s_v[r + k, sl] + a_v[r + k, sl]
                         + b_v[r + k, sl] + c_v[r + k, sl])
                    o_v[r + k, sl] = jnp.maximum(v, 0.0)
            return c2

        lax.fori_loop(0, R // 2, pair_rows, 0, unroll=False)

    for sub in (0, 1):
        for d in idx_copies(sub, bufs[sub]):
            d.start()
    for sub in (0, 1):
        for d in idx_copies(sub, bufs[sub]):
            d.wait()
        for d in in_copies(sub, bufs[sub]):
            d.start()

    def pair_body(t, carry):
        for sub in (0, 1):
            buf = bufs[sub]
            ci = 2 * t + sub
            for d in in_copies(ci, buf):
                d.wait()

            @pl.when(ci + 2 < NCHUNK)
            def _():
                for d in idx_copies(ci + 2, buf):
                    d.start()

            @pl.when(t > 0)
            def _():
                out_copy(ci - 2, buf).wait()

            compute(buf)
            out_copy(ci, buf).start()

            @pl.when(ci + 2 < NCHUNK)
            def _():
                for d in idx_copies(ci + 2, buf):
                    d.wait()
                for d in in_copies(ci + 2, buf):
                    d.start()
        return carry

    lax.fori_loop(0, NPAIR, pair_body, 0, unroll=False)
    out_copy(NCHUNK - 2, bufs[0]).wait()
    out_copy(NCHUNK - 1, bufs[1]).wait()


def _sc_combine(s, a, b, c, idx_n, idx_p, idx_m):
    mesh = plsc.VectorSubcoreMesh(core_axis_name="c", subcore_axis_name="s")
    rows_f32 = pltpu.VMEM((R, D), jnp.float32)
    idx_t = pltpu.VMEM((R,), jnp.int32)
    fn = pl.kernel(
        _sc_body,
        out_type=jax.ShapeDtypeStruct((N, D), jnp.float32),
        mesh=mesh,
        scratch_types=(
            [rows_f32] * 10
            + [idx_t] * 6
            + [pltpu.SemaphoreType.DMA] * 6
        ),
    )
    return fn(s, a, b, c, idx_n, idx_p, idx_m)


def kernel(features, next_indices, prev_indices, mate_indices, face_indices,
           W_self, b_self, W_next, b_next, W_prev, b_prev, W_mate, b_mate):
    del face_indices

    w_stacked = jnp.stack([W_self.T, W_next.T, W_prev.T, W_mate.T])
    b_total = (b_self + b_next + b_prev + b_mate).reshape(1, D)
    s, a, b, c = _tc_transform(features, w_stacked, b_total)
    return _sc_combine(
        s, a, b, c,
        next_indices.astype(jnp.int32),
        prev_indices.astype(jnp.int32),
        mate_indices.astype(jnp.int32),
    )


# speedup vs baseline: 1.0996x; 1.0003x over previous
import jax
import jax.numpy as jnp
from jax import lax
from jax.experimental import pallas as pl
from jax.experimental.pallas import tpu as pltpu
from jax.experimental.pallas import tpu_sc as plsc

N = 320000
D = 128


TC_BLK = 10000


def _tc_body(x_ref, w_ref, b_ref, s_ref, a_ref, bb_ref, c_ref):
    x = x_ref[...]
    s_ref[...] = jnp.dot(x, w_ref[0], preferred_element_type=jnp.float32) + b_ref[...]
    a_ref[...] = jnp.dot(x, w_ref[1], preferred_element_type=jnp.float32)
    bb_ref[...] = jnp.dot(x, w_ref[2], preferred_element_type=jnp.float32)
    c_ref[...] = jnp.dot(x, w_ref[3], preferred_element_type=jnp.float32)


def _tc_transform(x, w_stacked, b_total):
    row_spec = pl.BlockSpec((TC_BLK, D), lambda i: (i, 0))
    return pl.pallas_call(
        _tc_body,
        grid=(N // TC_BLK,),
        in_specs=[
            row_spec,
            pl.BlockSpec((4, D, D), lambda i: (0, 0, 0)),
            pl.BlockSpec((1, D), lambda i: (0, 0)),
        ],
        out_specs=[row_spec, row_spec, row_spec, row_spec],
        out_shape=[
            jax.ShapeDtypeStruct((N, D), jnp.float32),
            jax.ShapeDtypeStruct((N, D), jnp.float32),
            jax.ShapeDtypeStruct((N, D), jnp.float32),
            jax.ShapeDtypeStruct((N, D), jnp.float32),
        ],
    )(x, w_stacked, b_total)


NC = 2
NS = 16
NW = NC * NS
PW = N // NW
R = 40
NCHUNK = PW // R
NPAIR = NCHUNK // 2


def _sc_body(s_hbm, a_hbm, b_hbm, c_hbm, in_hbm, ip_hbm, im_hbm, out_hbm,
             s0, a0, b0, c0, o0, s1, a1, b1, c1, o1,
             in0, ip0, im0, in1, ip1, im1,
             semi0, semo0, semx0, semi1, semo1, semx1):
    wid = lax.axis_index("s") * NC + lax.axis_index("c")
    base0 = wid * PW
    bufs = ((s0, a0, b0, c0, o0, in0, ip0, im0, semi0, semo0, semx0),
            (s1, a1, b1, c1, o1, in1, ip1, im1, semi1, semo1, semx1))

    def idx_copies(ci, buf):
        in_v, ip_v, im_v, semx = buf[5], buf[6], buf[7], buf[10]
        base = base0 + ci * R
        return (
            pltpu.make_async_copy(in_hbm.at[pl.ds(base, R)], in_v, semx),
            pltpu.make_async_copy(ip_hbm.at[pl.ds(base, R)], ip_v, semx),
            pltpu.make_async_copy(im_hbm.at[pl.ds(base, R)], im_v, semx),
        )

    def in_copies(ci, buf):
        s_v, a_v, b_v, c_v = buf[:4]
        in_v, ip_v, im_v, semi = buf[5], buf[6], buf[7], buf[8]
        base = base0 + ci * R
        return (
            pltpu.make_async_copy(a_hbm.at[in_v], a_v, semi),
            pltpu.make_async_copy(b_hbm.at[ip_v], b_v, semi),
            pltpu.make_async_copy(c_hbm.at[im_v], c_v, semi),
            pltpu.make_async_copy(s_hbm.at[pl.ds(base, R)], s_v, semi),
        )

    def out_copy(ci, buf):
        o_v, semo = buf[4], buf[9]
        return pltpu.make_async_copy(o_v, out_hbm.at[pl.ds(base0 + ci * R, R)], semo)

    def compute(buf):
        s_v, a_v, b_v, c_v, o_v = buf[:5]

        def pair_rows(r2, c2):
            r = 2 * r2
            for k in (0, 1):
                for j in range(D // 16):
                    sl = pl.ds(j * 16, 16)
                    v = (s_v[r + k, sl] + a_v[r + k, sl]
                         + b_v[r + k, sl] + c_v[r + k, sl])
                    o_v[r + k, sl] = jnp.maximum(v, 0.0)
            return c2

        lax.fori_loop(0, R // 2, pair_rows, 0, unroll=False)

    for sub in (0, 1):
        for d in idx_copies(sub, bufs[sub]):
            d.start()
    for sub in (0, 1):
        for d in idx_copies(sub, bufs[sub]):
            d.wait()
        for d in in_copies(sub, bufs[sub]):
            d.start()

    def pair_body(t, carry):
        for sub in (0, 1):
            buf = bufs[sub]
            ci = 2 * t + sub
            for d in in_copies(ci, buf):
                d.wait()

            @pl.when(ci + 2 < NCHUNK)
            def _():
                for d in idx_copies(ci + 2, buf):
                    d.start()

            @pl.when(t > 0)
            def _():
                out_copy(ci - 2, buf).wait()

            compute(buf)
            out_copy(ci, buf).start()

            @pl.when(ci + 2 < NCHUNK)
            def _():
                for d in idx_copies(ci + 2, buf):
                    d.wait()
                for d in in_copies(ci + 2, buf):
                    d.start()
        return carry

    lax.fori_loop(0, NPAIR, pair_body, 0, unroll=False)
    out_copy(NCHUNK - 2, bufs[0]).wait()
    out_copy(NCHUNK - 1, bufs[1]).wait()


def _sc_combine(s, a, b, c, idx_n, idx_p, idx_m):
    mesh = plsc.VectorSubcoreMesh(core_axis_name="c", subcore_axis_name="s")
    rows_f32 = pltpu.VMEM((R, D), jnp.float32)
    idx_t = pltpu.VMEM((R,), jnp.int32)
    fn = pl.kernel(
        _sc_body,
        out_type=jax.ShapeDtypeStruct((N, D), jnp.float32),
        mesh=mesh,
        scratch_types=(
            [rows_f32] * 10
            + [idx_t] * 6
            + [pltpu.SemaphoreType.DMA] * 6
        ),
    )
    return fn(s, a, b, c, idx_n, idx_p, idx_m)


def kernel(features, next_indices, prev_indices, mate_indices, face_indices,
           W_self, b_self, W_next, b_next, W_prev, b_prev, W_mate, b_mate):
    del face_indices

    w_stacked = jnp.stack([W_self.T, W_next.T, W_prev.T, W_mate.T])
    b_total = (b_self + b_next + b_prev + b_mate).reshape(1, D)
    s, a, b, c = _tc_transform(features, w_stacked, b_total)
    return _sc_combine(
        s, a, b, c,
        next_indices.astype(jnp.int32),
        prev_indices.astype(jnp.int32),
        mate_indices.astype(jnp.int32),
    )
